# baseline (speedup 1.0000x reference)
import jax
import jax.numpy as jnp
from jax import lax
from jax.experimental import pallas as pl
from jax.experimental.pallas import tpu as pltpu
from jax.experimental.pallas import tpu_sc as plsc

N_PIX = 8192
EMB = 1024
B = 4

NC = 2
NS = 16
NW = NC * NS

R = 16
CH = R * EMB
ROWS_PER_W = N_PIX // NW
CHUNKS_PER_W = ROWS_PER_W // R
XSTRIDE = N_PIX * EMB


def _sc_body(x_hbm, pos_hbm, out_hbm, pos_v, x0, x1, x2, x3):
    wid = lax.axis_index("s") * NC + lax.axis_index("c")
    xbufs = (x0, x1, x2, x3)

    def chunk(c, _):
        word0 = (wid * ROWS_PER_W + c * R) * EMB
        pltpu.sync_copy(pos_hbm.at[pl.ds(word0, CH)], pos_v)
        for b in range(B):
            pltpu.sync_copy(x_hbm.at[pl.ds(b * XSTRIDE + word0, CH)], xbufs[b])

        def vec(i, _):
            off = i * 16
            pv = pos_v[pl.ds(off, 16)]
            for b in range(B):
                xbufs[b][pl.ds(off, 16)] = xbufs[b][pl.ds(off, 16)] + pv
            return ()

        lax.fori_loop(0, CH // 16, vec, (), unroll=4)
        for b in range(B):
            pltpu.sync_copy(xbufs[b], out_hbm.at[pl.ds(b * XSTRIDE + word0, CH)])
        return ()

    lax.fori_loop(0, CHUNKS_PER_W, chunk, ())


def _sc_add(x_flat, pos_flat):
    mesh = plsc.VectorSubcoreMesh(core_axis_name="c", subcore_axis_name="s")
    f = pl.kernel(
        _sc_body,
        out_type=jax.ShapeDtypeStruct((B * XSTRIDE,), jnp.float32),
        mesh=mesh,
        scratch_types=[pltpu.VMEM((CH,), jnp.float32)] * 5,
    )
    return f(x_flat, pos_flat)


def kernel(x, pos_table):
    out = _sc_add(x.reshape(-1), pos_table.reshape(-1))
    return out.reshape(B, N_PIX, EMB)


# speedup vs baseline: 1.3936x; 1.3936x over previous
import jax
import jax.numpy as jnp
from jax import lax
from jax.experimental import pallas as pl
from jax.experimental.pallas import tpu as pltpu
from jax.experimental.pallas import tpu_sc as plsc

N_PIX = 8192
EMB = 1024
B = 4

NC = 2
NS = 16
NW = NC * NS

R = 8
CH = R * EMB
ROWS_PER_W = N_PIX // NW
NCH = ROWS_PER_W // R
XSTRIDE = N_PIX * EMB


def _sc_body(x_hbm, pos_hbm, out_hbm, p0, p1, xb0, xb1, si0, si1, so0, so1):
    wid = lax.axis_index("s") * NC + lax.axis_index("c")
    base = wid * ROWS_PER_W * EMB
    pbufs = (p0, p1)
    xbufs = (xb0, xb1)
    isems = (si0, si1)
    osems = (so0, so1)

    def in_copies(c, s):
        word0 = base + c * CH
        return (
            pltpu.make_async_copy(pos_hbm.at[pl.ds(word0, CH)], pbufs[s], isems[s]),
            pltpu.make_async_copy(x_hbm.at[:, pl.ds(word0, CH)], xbufs[s], isems[s]),
        )

    def out_copy(c, s):
        word0 = base + c * CH
        return pltpu.make_async_copy(xbufs[s], out_hbm.at[:, pl.ds(word0, CH)], osems[s])

    def start_in(c, s):
        for cp in in_copies(c, s):
            cp.start()

    def wait_in(c, s):
        for cp in in_copies(c, s):
            cp.wait()

    def compute(s):
        pv_ref, xv_ref = pbufs[s], xbufs[s]

        def vec(i, _):
            off = i * 16
            pv = pv_ref[pl.ds(off, 16)]
            for b in range(B):
                xv_ref[b, pl.ds(off, 16)] = xv_ref[b, pl.ds(off, 16)] + pv
            return ()

        lax.fori_loop(0, CH // 16, vec, (), unroll=4)

    start_in(0, 0)
    start_in(1, 1)
    wait_in(0, 0)
    compute(0)
    out_copy(0, 0).start()

    def pair(k, _):
        for s in (1, 0):
            c = 2 * k + 1 + (1 - s)
            out_copy(c - 1, 1 - s).wait()
            start_in(c + 1, 1 - s)
            wait_in(c, s)
            compute(s)
            out_copy(c, s).start()
        return ()

    lax.fori_loop(0, (NCH - 2) // 2, pair, ())

    c = NCH - 1
    out_copy(c - 1, 0).wait()
    wait_in(c, 1)
    compute(1)
    out_copy(c, 1).start()
    out_copy(c, 1).wait()


def _sc_add(x2d, pos_flat):
    mesh = plsc.VectorSubcoreMesh(core_axis_name="c", subcore_axis_name="s")
    f = pl.kernel(
        _sc_body,
        out_type=jax.ShapeDtypeStruct((B, XSTRIDE), jnp.float32),
        mesh=mesh,
        scratch_types=[
            pltpu.VMEM((CH,), jnp.float32),
            pltpu.VMEM((CH,), jnp.float32),
            pltpu.VMEM((B, CH), jnp.float32),
            pltpu.VMEM((B, CH), jnp.float32),
            pltpu.SemaphoreType.DMA,
            pltpu.SemaphoreType.DMA,
            pltpu.SemaphoreType.DMA,
            pltpu.SemaphoreType.DMA,
        ],
    )
    return f(x2d, pos_flat)


def kernel(x, pos_table):
    out = _sc_add(x.reshape(B, XSTRIDE), pos_table.reshape(-1))
    return out.reshape(B, N_PIX, EMB)


# speedup vs baseline: 4.3608x; 3.1291x over previous
import jax
import jax.numpy as jnp
from jax import lax
from jax.experimental import pallas as pl
from jax.experimental.pallas import tpu as pltpu
from jax.experimental.pallas import tpu_sc as plsc

N_PIX = 8192
EMB = 1024
B = 4

NC = 2
NS = 16
NW = NC * NS

R = 8
ROWS_PER_W = N_PIX // NW
NCH = ROWS_PER_W // R


def _sc_body(x_hbm, pos_hbm, out_hbm, p0, p1, xb0, xb1, si0, si1, so0, so1):
    wid = lax.axis_index("s") * NC + lax.axis_index("c")
    row_base = wid * ROWS_PER_W
    pbufs = (p0, p1)
    xbufs = (xb0, xb1)
    isems = (si0, si1)
    osems = (so0, so1)

    def in_copies(c, s):
        r0 = row_base + c * R
        return (
            pltpu.make_async_copy(pos_hbm.at[pl.ds(r0, R), :], pbufs[s], isems[s]),
            pltpu.make_async_copy(x_hbm.at[:, pl.ds(r0, R), :], xbufs[s], isems[s]),
        )

    def out_copy(c, s):
        r0 = row_base + c * R
        return pltpu.make_async_copy(
            xbufs[s], out_hbm.at[:, pl.ds(r0, R), :], osems[s])

    def start_in(c, s):
        for cp in in_copies(c, s):
            cp.start()

    def wait_in(c, s):
        for cp in in_copies(c, s):
            cp.wait()

    def compute(s):
        pv_ref, xv_ref = pbufs[s], xbufs[s]

        def vec(i, _):
            r = i // (EMB // 16)
            off = (i % (EMB // 16)) * 16
            pv = pv_ref[r, pl.ds(off, 16)]
            for b in range(B):
                xv_ref[b, r, pl.ds(off, 16)] = xv_ref[b, r, pl.ds(off, 16)] + pv
            return ()

        lax.fori_loop(0, R * EMB // 16, vec, (), unroll=4)

    start_in(0, 0)
    start_in(1, 1)
    wait_in(0, 0)
    compute(0)
    out_copy(0, 0).start()

    def pair(k, _):
        for s in (1, 0):
            c = 2 * k + 1 + (1 - s)
            out_copy(c - 1, 1 - s).wait()
            start_in(c + 1, 1 - s)
            wait_in(c, s)
            compute(s)
            out_copy(c, s).start()
        return ()

    lax.fori_loop(0, (NCH - 2) // 2, pair, ())

    c = NCH - 1
    out_copy(c - 1, 0).wait()
    wait_in(c, 1)
    compute(1)
    out_copy(c, 1).start()
    out_copy(c, 1).wait()


def _sc_add(x, pos_table):
    mesh = plsc.VectorSubcoreMesh(core_axis_name="c", subcore_axis_name="s")
    f = pl.kernel(
        _sc_body,
        out_type=jax.ShapeDtypeStruct((B, N_PIX, EMB), jnp.float32),
        mesh=mesh,
        scratch_types=[
            pltpu.VMEM((R, EMB), jnp.float32),
            pltpu.VMEM((R, EMB), jnp.float32),
            pltpu.VMEM((B, R, EMB), jnp.float32),
            pltpu.VMEM((B, R, EMB), jnp.float32),
            pltpu.SemaphoreType.DMA,
            pltpu.SemaphoreType.DMA,
            pltpu.SemaphoreType.DMA,
            pltpu.SemaphoreType.DMA,
        ],
    )
    return f(x, pos_table)


def kernel(x, pos_table):
    return _sc_add(x, pos_table)
